# baseline (speedup 1.0000x reference)
import functools

import jax
import jax.numpy as jnp
from jax import lax
from jax.experimental import pallas as pl
from jax.experimental.pallas import tpu as pltpu
from jax.experimental.pallas import tpu_sc as plsc

N = 10000
E = 320000
D = 128

NC, NS, L = 2, 16, 16
NW = NC * NS
CHUNK = 128
N_CHUNKS = (2 * E) // CHUNK
MAX_K = (N_CHUNKS + NW - 1) // NW
CPAD = CHUNK + 1


def _sc_gather_mul_t(srcs, dsts, x):
    mesh = plsc.VectorSubcoreMesh(core_axis_name="c", subcore_axis_name="s")
    f32 = jnp.float32

    @functools.partial(
        pl.kernel,
        out_type=jax.ShapeDtypeStruct((D, 2 * E), f32),
        mesh=mesh,
        scratch_types=[
            [pltpu.VMEM((CHUNK,), jnp.int32)] * 2,
            [pltpu.VMEM((CHUNK,), jnp.int32)] * 2,
            [pltpu.VMEM((CHUNK, D), f32)] * 2,
            [pltpu.VMEM((CHUNK, D), f32)] * 2,
            pltpu.VMEM((D, CPAD), f32),
            [pltpu.SemaphoreType.DMA] * 2,
            pltpu.SemaphoreType.DMA,
            [pltpu.SemaphoreType.DMA] * 2,
        ],
        compiler_params=pltpu.CompilerParams(needs_layout_passes=False),
    )
    def k(srcs_hbm, dsts_hbm, x_hbm, ht_hbm, idx_s, idx_d, a, b, ot,
          sem_g, sem_w, sem_i):
        wid = lax.axis_index("s") * NC + lax.axis_index("c")

        def cid(kk):
            return wid + kk * NW

        def fire_idx(kk, q):
            base = cid(kk) * CHUNK
            pltpu.async_copy(srcs_hbm.at[pl.ds(base, CHUNK)], idx_s[q],
                             sem_i[q])
            pltpu.async_copy(dsts_hbm.at[pl.ds(base, CHUNK)], idx_d[q],
                             sem_i[q])

        def wait_idx(q):
            pltpu.make_async_copy(srcs_hbm.at[pl.ds(0, CHUNK)], idx_s[q],
                                  sem_i[q]).wait()
            pltpu.make_async_copy(dsts_hbm.at[pl.ds(0, CHUNK)], idx_d[q],
                                  sem_i[q]).wait()

        def fire_gather(q):
            pltpu.async_copy(x_hbm.at[idx_s[q]], a[q], sem_g[q])
            pltpu.async_copy(x_hbm.at[idx_d[q]], b[q], sem_g[q])

        def wait_gather(q):
            pltpu.make_async_copy(x_hbm.at[idx_s[q]], a[q], sem_g[q]).wait()
            pltpu.make_async_copy(x_hbm.at[idx_d[q]], b[q], sem_g[q]).wait()

        fire_idx(0, 0)
        wait_idx(0)
        fire_gather(0)
        fire_idx(1, 1)
        iota = lax.iota(jnp.int32, L)

        @pl.loop(0, MAX_K + (MAX_K % 2), step=2)
        def chunk_pair(k0):
            for p in range(2):
                kk = k0 + p
                have = cid(kk) < N_CHUNKS
                have_n1 = cid(kk + 1) < N_CHUNKS
                have_n2 = cid(kk + 2) < N_CHUNKS

                @pl.when(have_n1)
                def _():
                    wait_idx(1 - p)
                    fire_gather(1 - p)

                @pl.when(have)
                def _():
                    wait_gather(p)

                @pl.when(have_n2)
                def _():
                    fire_idx(kk + 2, p)

                @pl.when(have)
                def _():
                    @pl.when(kk >= 1)
                    def _():
                        pltpu.make_async_copy(
                            ot.at[:, pl.ds(0, CHUNK)],
                            ht_hbm.at[:, pl.ds(0, CHUNK)],
                            sem_w).wait()

                    @plsc.parallel_loop(0, CHUNK, 1, unroll=4)
                    def row_body(r):
                        col = jnp.full((L,), 0, jnp.int32) + r
                        for j in range(D // L):
                            sl = pl.ds(j * L, L)
                            prod = a[p][r, sl] * b[p][r, sl]
                            plsc.store_scatter(ot, [iota + j * L, col],
                                               prod)

                    pltpu.async_copy(
                        ot.at[:, pl.ds(0, CHUNK)],
                        ht_hbm.at[:, pl.ds(cid(kk) * CHUNK, CHUNK)],
                        sem_w)

        @pl.when(wid < N_CHUNKS)
        def _():
            pltpu.make_async_copy(ot.at[:, pl.ds(0, CHUNK)],
                                  ht_hbm.at[:, pl.ds(0, CHUNK)],
                                  sem_w).wait()

    return k(srcs, dsts, x)


B_TC = 2560
NBLK = E // B_TC


def _tc_body(hpT_ref, hnT_ref, w1_ref, b1_ref, w2_ref, b2_ref, oT_ref):
    catT = jnp.concatenate(
        [jnp.maximum(hpT_ref[...], 0.0), jnp.maximum(hnT_ref[...], 0.0)],
        axis=0)
    zT = jnp.maximum(
        jnp.dot(w1_ref[...], catT, preferred_element_type=jnp.float32)
        + b1_ref[...], 0.0)
    oT_ref[...] = jax.nn.sigmoid(
        jnp.dot(w2_ref[...], zT, preferred_element_type=jnp.float32)
        + b2_ref[...])


def _tc_mlp_t(hT, Wb1T, bb1T, Wb2T, bb2T):
    return pl.pallas_call(
        _tc_body,
        grid=(NBLK,),
        in_specs=[
            pl.BlockSpec((D, B_TC), lambda i: (0, i)),
            pl.BlockSpec((D, B_TC), lambda i: (0, i + NBLK)),
            pl.BlockSpec((3 * D, 2 * D), lambda i: (0, 0)),
            pl.BlockSpec((3 * D, 1), lambda i: (0, 0)),
            pl.BlockSpec((8, 3 * D), lambda i: (0, 0)),
            pl.BlockSpec((8, 1), lambda i: (0, 0)),
        ],
        out_specs=pl.BlockSpec((8, B_TC), lambda i: (0, i)),
        out_shape=jax.ShapeDtypeStruct((8, E), jnp.float32),
    )(hT, hT, Wb1T, bb1T, Wb2T, bb2T)


@jax.jit
def kernel(x, edge_index, edge_index_neg, W1, b1, W2, b2, We1, be1, We2, be2):
    srcs = jnp.concatenate([edge_index[0], edge_index_neg[0]])
    dsts = jnp.concatenate([edge_index[1], edge_index_neg[1]])

    hT = _sc_gather_mul_t(srcs, dsts, x)

    f32 = jnp.float32
    Wb1 = jnp.zeros((2 * D, 3 * D), f32)
    Wb1 = Wb1.at[:D, :D].set(W1).at[:D, D:2 * D].set(We1).at[D:, 2 * D:].set(We1)
    bb1T = jnp.concatenate([b1, be1, be1]).reshape(3 * D, 1)
    Wb2 = jnp.zeros((3 * D, 8), f32)
    Wb2 = Wb2.at[:D, :4].set(W2).at[D:2 * D, 4:5].set(We2).at[2 * D:, 5:6].set(We2)
    bb2T = jnp.concatenate([b2, be2, be2, jnp.zeros((2,), f32)]).reshape(8, 1)

    oT = _tc_mlp_t(hT, Wb1.T, bb1T, Wb2.T, bb2T)
    return oT[:4].T, oT[4], oT[5]


# speedup vs baseline: 2.3509x; 2.3509x over previous
import functools

import jax
import jax.numpy as jnp
from jax import lax
from jax.experimental import pallas as pl
from jax.experimental.pallas import tpu as pltpu
from jax.experimental.pallas import tpu_sc as plsc

N = 10000
E = 320000
D = 128

NC, NS, L = 2, 16, 16
NW = NC * NS
ROWS_PER_W = (2 * E) // NW
CHUNK = 80
N_CHUNKS = ROWS_PER_W // CHUNK


def _sc_gather_mul(srcs, dsts, x):
    mesh = plsc.VectorSubcoreMesh(core_axis_name="c", subcore_axis_name="s")
    f32 = jnp.float32

    @functools.partial(
        pl.kernel,
        out_type=jax.ShapeDtypeStruct((2 * E, D), f32),
        mesh=mesh,
        scratch_types=[
            pltpu.VMEM((ROWS_PER_W,), jnp.int32),
            pltpu.VMEM((ROWS_PER_W,), jnp.int32),
            [pltpu.VMEM((CHUNK, D), f32)] * 2,
            [pltpu.VMEM((CHUNK, D), f32)] * 2,
            [pltpu.VMEM((CHUNK, D), f32)] * 2,
            [pltpu.SemaphoreType.DMA] * 2,
            [pltpu.SemaphoreType.DMA] * 2,
            pltpu.SemaphoreType.DMA,
        ],
    )
    def k(srcs_hbm, dsts_hbm, x_hbm, h_hbm, idx_s, idx_d, a, b, o,
          sem_g, sem_w, sem_i):
        wid = lax.axis_index("s") * NC + lax.axis_index("c")
        w_base = wid * ROWS_PER_W

        ci = pltpu.async_copy(srcs_hbm.at[pl.ds(w_base, ROWS_PER_W)], idx_s,
                              sem_i)
        cd = pltpu.async_copy(dsts_hbm.at[pl.ds(w_base, ROWS_PER_W)], idx_d,
                              sem_i)
        ci.wait()
        cd.wait()

        def fire(kk, p):
            pltpu.async_copy(x_hbm.at[idx_s.at[pl.ds(kk * CHUNK, CHUNK)]],
                             a[p], sem_g[p])
            pltpu.async_copy(x_hbm.at[idx_d.at[pl.ds(kk * CHUNK, CHUNK)]],
                             b[p], sem_g[p])

        fire(0, 0)

        @pl.loop(0, N_CHUNKS, step=2)
        def chunk_pair(k0):
            for p in range(2):
                kk = k0 + p

                @pl.when(kk + 1 < N_CHUNKS)
                def _():
                    fire(kk + 1, 1 - p)

                pltpu.make_async_copy(x_hbm.at[idx_s.at[pl.ds(0, CHUNK)]],
                                      a[p], sem_g[p]).wait()
                pltpu.make_async_copy(x_hbm.at[idx_d.at[pl.ds(0, CHUNK)]],
                                      b[p], sem_g[p]).wait()

                @pl.when(kk >= 2)
                def _():
                    pltpu.make_async_copy(
                        o[p], h_hbm.at[pl.ds(0, CHUNK)],
                        sem_w[p]).wait()

                @plsc.parallel_loop(0, CHUNK, 1, unroll=4)
                def row_body(r):
                    for j in range(D // L):
                        sl = pl.ds(j * L, L)
                        o[p][r, sl] = a[p][r, sl] * b[p][r, sl]

                pltpu.async_copy(
                    o[p], h_hbm.at[pl.ds(w_base + kk * CHUNK, CHUNK)],
                    sem_w[p])

        for p in range(2):
            pltpu.make_async_copy(o[p], h_hbm.at[pl.ds(0, CHUNK)],
                                  sem_w[p]).wait()

    return k(srcs, dsts, x)


B_TC = 2560
NBLK = E // B_TC


def _tc_body(hp_ref, hn_ref, w1_ref, b1_ref, w2_ref, b2_ref, oT_ref):
    cat = jnp.concatenate(
        [jnp.maximum(hp_ref[...], 0.0), jnp.maximum(hn_ref[...], 0.0)], axis=1)
    z = jnp.maximum(
        jnp.dot(cat, w1_ref[...], preferred_element_type=jnp.float32)
        + b1_ref[...], 0.0)
    o = jax.nn.sigmoid(
        jnp.dot(z, w2_ref[...], preferred_element_type=jnp.float32)
        + b2_ref[...])
    oT_ref[...] = o.T


def _tc_mlp(h2, Wb1, bb1, Wb2, bb2):
    return pl.pallas_call(
        _tc_body,
        grid=(NBLK,),
        in_specs=[
            pl.BlockSpec((B_TC, D), lambda i: (i, 0)),
            pl.BlockSpec((B_TC, D), lambda i: (i + NBLK, 0)),
            pl.BlockSpec((2 * D, 3 * D), lambda i: (0, 0)),
            pl.BlockSpec((1, 3 * D), lambda i: (0, 0)),
            pl.BlockSpec((3 * D, 8), lambda i: (0, 0)),
            pl.BlockSpec((1, 8), lambda i: (0, 0)),
        ],
        out_specs=pl.BlockSpec((8, B_TC), lambda i: (0, i)),
        out_shape=jax.ShapeDtypeStruct((8, E), jnp.float32),
    )(h2, h2, Wb1, bb1, Wb2, bb2)


@jax.jit
def kernel(x, edge_index, edge_index_neg, W1, b1, W2, b2, We1, be1, We2, be2):
    srcs = jnp.concatenate([edge_index[0], edge_index_neg[0]])
    dsts = jnp.concatenate([edge_index[1], edge_index_neg[1]])

    h2 = _sc_gather_mul(srcs, dsts, x)

    f32 = jnp.float32
    Wb1 = jnp.zeros((2 * D, 3 * D), f32)
    Wb1 = Wb1.at[:D, :D].set(W1).at[:D, D:2 * D].set(We1).at[D:, 2 * D:].set(We1)
    bb1 = jnp.concatenate([b1, be1, be1]).reshape(1, 3 * D)
    Wb2 = jnp.zeros((3 * D, 8), f32)
    Wb2 = Wb2.at[:D, :4].set(W2).at[D:2 * D, 4:5].set(We2).at[2 * D:, 5:6].set(We2)
    bb2 = jnp.concatenate([b2, be2, be2, jnp.zeros((2,), f32)]).reshape(1, 8)

    oT = _tc_mlp(h2, Wb1, bb1, Wb2, bb2)
    return oT[:4].T, oT[4], oT[5]


# speedup vs baseline: 2.3513x; 1.0002x over previous
import functools

import jax
import jax.numpy as jnp
from jax import lax
from jax.experimental import pallas as pl
from jax.experimental.pallas import tpu as pltpu
from jax.experimental.pallas import tpu_sc as plsc

N = 10000
E = 320000
D = 128

NC, NS, L = 2, 16, 16
NW = NC * NS
ROWS_PER_W = (2 * E) // NW
CHUNK = 80
N_CHUNKS = ROWS_PER_W // CHUNK


def _sc_gather_mul(srcs, dsts, x):
    mesh = plsc.VectorSubcoreMesh(core_axis_name="c", subcore_axis_name="s")
    f32 = jnp.float32

    @functools.partial(
        pl.kernel,
        out_type=jax.ShapeDtypeStruct((2 * E, D), f32),
        mesh=mesh,
        scratch_types=[
            pltpu.VMEM((ROWS_PER_W,), jnp.int32),
            pltpu.VMEM((ROWS_PER_W,), jnp.int32),
            [pltpu.VMEM((CHUNK, D), f32)] * 2,
            [pltpu.VMEM((CHUNK, D), f32)] * 2,
            [pltpu.VMEM((CHUNK, D), f32)] * 2,
            [pltpu.SemaphoreType.DMA] * 2,
            [pltpu.SemaphoreType.DMA] * 2,
            pltpu.SemaphoreType.DMA,
        ],
    )
    def k(srcs_hbm, dsts_hbm, x_hbm, h_hbm, idx_s, idx_d, a, b, o,
          sem_g, sem_w, sem_i):
        wid = lax.axis_index("s") * NC + lax.axis_index("c")
        w_base = wid * ROWS_PER_W

        ci = pltpu.async_copy(srcs_hbm.at[pl.ds(w_base, ROWS_PER_W)], idx_s,
                              sem_i)
        cd = pltpu.async_copy(dsts_hbm.at[pl.ds(w_base, ROWS_PER_W)], idx_d,
                              sem_i)
        ci.wait()
        cd.wait()

        def fire(kk, p):
            pltpu.async_copy(x_hbm.at[idx_s.at[pl.ds(kk * CHUNK, CHUNK)]],
                             a[p], sem_g[p])
            pltpu.async_copy(x_hbm.at[idx_d.at[pl.ds(kk * CHUNK, CHUNK)]],
                             b[p], sem_g[p])

        fire(0, 0)

        @pl.loop(0, N_CHUNKS, step=2)
        def chunk_pair(k0):
            for p in range(2):
                kk = k0 + p

                @pl.when(kk + 1 < N_CHUNKS)
                def _():
                    fire(kk + 1, 1 - p)

                pltpu.make_async_copy(x_hbm.at[idx_s.at[pl.ds(0, CHUNK)]],
                                      a[p], sem_g[p]).wait()
                pltpu.make_async_copy(x_hbm.at[idx_d.at[pl.ds(0, CHUNK)]],
                                      b[p], sem_g[p]).wait()

                @pl.when(kk >= 2)
                def _():
                    pltpu.make_async_copy(
                        o[p], h_hbm.at[pl.ds(0, CHUNK)],
                        sem_w[p]).wait()

                @plsc.parallel_loop(0, CHUNK, 1, unroll=4)
                def row_body(r):
                    for j in range(D // L):
                        sl = pl.ds(j * L, L)
                        o[p][r, sl] = a[p][r, sl] * b[p][r, sl]

                pltpu.async_copy(
                    o[p], h_hbm.at[pl.ds(w_base + kk * CHUNK, CHUNK)],
                    sem_w[p])

        for p in range(2):
            pltpu.make_async_copy(o[p], h_hbm.at[pl.ds(0, CHUNK)],
                                  sem_w[p]).wait()

    return k(srcs, dsts, x)


B_TC = 2560
NBLK = E // B_TC


def _tc_body(hp_ref, hn_ref, w1_ref, b1_ref, w2_ref, b2_ref, oT_ref):
    cat = jnp.concatenate(
        [jnp.maximum(hp_ref[...], 0.0), jnp.maximum(hn_ref[...], 0.0)], axis=1)
    z = jnp.maximum(
        jnp.dot(cat.astype(jnp.bfloat16), w1_ref[...],
                preferred_element_type=jnp.float32)
        + b1_ref[...], 0.0)
    o = jax.nn.sigmoid(
        jnp.dot(z, w2_ref[...], preferred_element_type=jnp.float32)
        + b2_ref[...])
    oT_ref[...] = o.T


def _tc_mlp(h2, Wb1, bb1, Wb2, bb2):
    return pl.pallas_call(
        _tc_body,
        grid=(NBLK,),
        in_specs=[
            pl.BlockSpec((B_TC, D), lambda i: (i, 0)),
            pl.BlockSpec((B_TC, D), lambda i: (i + NBLK, 0)),
            pl.BlockSpec((2 * D, 3 * D), lambda i: (0, 0)),
            pl.BlockSpec((1, 3 * D), lambda i: (0, 0)),
            pl.BlockSpec((3 * D, 8), lambda i: (0, 0)),
            pl.BlockSpec((1, 8), lambda i: (0, 0)),
        ],
        out_specs=pl.BlockSpec((8, B_TC), lambda i: (0, i)),
        out_shape=jax.ShapeDtypeStruct((8, E), jnp.float32),
    )(h2, h2, Wb1, bb1, Wb2, bb2)


@jax.jit
def kernel(x, edge_index, edge_index_neg, W1, b1, W2, b2, We1, be1, We2, be2):
    srcs = jnp.concatenate([edge_index[0], edge_index_neg[0]])
    dsts = jnp.concatenate([edge_index[1], edge_index_neg[1]])

    h2 = _sc_gather_mul(srcs, dsts, x)

    f32 = jnp.float32
    Wb1 = jnp.zeros((2 * D, 3 * D), f32)
    Wb1 = Wb1.at[:D, :D].set(W1).at[:D, D:2 * D].set(We1).at[D:, 2 * D:].set(We1)
    Wb1 = Wb1.astype(jnp.bfloat16)
    bb1 = jnp.concatenate([b1, be1, be1]).reshape(1, 3 * D)
    Wb2 = jnp.zeros((3 * D, 8), f32)
    Wb2 = Wb2.at[:D, :4].set(W2).at[D:2 * D, 4:5].set(We2).at[2 * D:, 5:6].set(We2)
    bb2 = jnp.concatenate([b2, be2, be2, jnp.zeros((2,), f32)]).reshape(1, 8)

    oT = _tc_mlp(h2, Wb1, bb1, Wb2, bb2)
    return oT[:4].T, oT[4], oT[5]


# speedup vs baseline: 2.4721x; 1.0513x over previous
import functools

import jax
import jax.numpy as jnp
from jax import lax
from jax.experimental import pallas as pl
from jax.experimental.pallas import tpu as pltpu
from jax.experimental.pallas import tpu_sc as plsc

N = 10000
E = 320000
D = 128

NC, NS, L = 2, 16, 16
NW = NC * NS
ROWS_PER_W = (2 * E) // NW
CHUNK = 80
N_CHUNKS = ROWS_PER_W // CHUNK


def _sc_gather_mul(srcs, dsts, x):
    mesh = plsc.VectorSubcoreMesh(core_axis_name="c", subcore_axis_name="s")
    f32 = jnp.float32

    @functools.partial(
        pl.kernel,
        out_type=jax.ShapeDtypeStruct((2 * E, D), f32),
        mesh=mesh,
        scratch_types=[
            pltpu.VMEM((ROWS_PER_W,), jnp.int32),
            pltpu.VMEM((ROWS_PER_W,), jnp.int32),
            [pltpu.VMEM((CHUNK, D), f32)] * 2,
            [pltpu.VMEM((CHUNK, D), f32)] * 2,
            [pltpu.VMEM((CHUNK, D), f32)] * 2,
            [pltpu.SemaphoreType.DMA] * 2,
            [pltpu.SemaphoreType.DMA] * 2,
            pltpu.SemaphoreType.DMA,
        ],
    )
    def k(srcs_hbm, dsts_hbm, x_hbm, h_hbm, idx_s, idx_d, a, b, o,
          sem_g, sem_w, sem_i):
        wid = lax.axis_index("s") * NC + lax.axis_index("c")
        w_base = wid * ROWS_PER_W

        ci = pltpu.async_copy(srcs_hbm.at[pl.ds(w_base, ROWS_PER_W)], idx_s,
                              sem_i)
        cd = pltpu.async_copy(dsts_hbm.at[pl.ds(w_base, ROWS_PER_W)], idx_d,
                              sem_i)
        ci.wait()
        cd.wait()

        def fire(kk, p):
            pltpu.async_copy(x_hbm.at[idx_s.at[pl.ds(kk * CHUNK, CHUNK)]],
                             a[p], sem_g[p])
            pltpu.async_copy(x_hbm.at[idx_d.at[pl.ds(kk * CHUNK, CHUNK)]],
                             b[p], sem_g[p])

        fire(0, 0)

        @pl.loop(0, N_CHUNKS, step=2)
        def chunk_pair(k0):
            for p in range(2):
                kk = k0 + p

                @pl.when(kk + 1 < N_CHUNKS)
                def _():
                    fire(kk + 1, 1 - p)

                pltpu.make_async_copy(x_hbm.at[idx_s.at[pl.ds(0, CHUNK)]],
                                      a[p], sem_g[p]).wait()
                pltpu.make_async_copy(x_hbm.at[idx_d.at[pl.ds(0, CHUNK)]],
                                      b[p], sem_g[p]).wait()

                @pl.when(kk >= 2)
                def _():
                    pltpu.make_async_copy(
                        o[p], h_hbm.at[pl.ds(0, CHUNK)],
                        sem_w[p]).wait()

                @plsc.parallel_loop(0, CHUNK, 1, unroll=4)
                def row_body(r):
                    for j in range(D // L):
                        sl = pl.ds(j * L, L)
                        o[p][r, sl] = a[p][r, sl] * b[p][r, sl]

                pltpu.async_copy(
                    o[p], h_hbm.at[pl.ds(w_base + kk * CHUNK, CHUNK)],
                    sem_w[p])

        for p in range(2):
            pltpu.make_async_copy(o[p], h_hbm.at[pl.ds(0, CHUNK)],
                                  sem_w[p]).wait()

    return k(srcs, dsts, x)


B_TC = 6400
NBLK = E // B_TC


def _tc_body(hp_ref, hn_ref, w1_ref, b1_ref, w2_ref, b2_ref, oT_ref):
    cat = jnp.concatenate(
        [jnp.maximum(hp_ref[...], 0.0), jnp.maximum(hn_ref[...], 0.0)], axis=1)
    z = jnp.maximum(
        jnp.dot(cat.astype(jnp.bfloat16), w1_ref[...],
                preferred_element_type=jnp.float32)
        + b1_ref[...], 0.0)
    o = jax.nn.sigmoid(
        jnp.dot(z, w2_ref[...], preferred_element_type=jnp.float32)
        + b2_ref[...])
    oT_ref[...] = o.T


def _tc_mlp(h2, Wb1, bb1, Wb2, bb2):
    return pl.pallas_call(
        _tc_body,
        grid=(NBLK,),
        in_specs=[
            pl.BlockSpec((B_TC, D), lambda i: (i, 0)),
            pl.BlockSpec((B_TC, D), lambda i: (i + NBLK, 0)),
            pl.BlockSpec((2 * D, 3 * D), lambda i: (0, 0)),
            pl.BlockSpec((1, 3 * D), lambda i: (0, 0)),
            pl.BlockSpec((3 * D, 8), lambda i: (0, 0)),
            pl.BlockSpec((1, 8), lambda i: (0, 0)),
        ],
        out_specs=pl.BlockSpec((8, B_TC), lambda i: (0, i)),
        out_shape=jax.ShapeDtypeStruct((8, E), jnp.float32),
    )(h2, h2, Wb1, bb1, Wb2, bb2)


@jax.jit
def kernel(x, edge_index, edge_index_neg, W1, b1, W2, b2, We1, be1, We2, be2):
    srcs = jnp.concatenate([edge_index[0], edge_index_neg[0]])
    dsts = jnp.concatenate([edge_index[1], edge_index_neg[1]])

    h2 = _sc_gather_mul(srcs, dsts, x)

    f32 = jnp.float32
    Wb1 = jnp.zeros((2 * D, 3 * D), f32)
    Wb1 = Wb1.at[:D, :D].set(W1).at[:D, D:2 * D].set(We1).at[D:, 2 * D:].set(We1)
    Wb1 = Wb1.astype(jnp.bfloat16)
    bb1 = jnp.concatenate([b1, be1, be1]).reshape(1, 3 * D)
    Wb2 = jnp.zeros((3 * D, 8), f32)
    Wb2 = Wb2.at[:D, :4].set(W2).at[D:2 * D, 4:5].set(We2).at[2 * D:, 5:6].set(We2)
    bb2 = jnp.concatenate([b2, be2, be2, jnp.zeros((2,), f32)]).reshape(1, 8)

    oT = _tc_mlp(h2, Wb1, bb1, Wb2, bb2)
    return oT[:4].T, oT[4], oT[5]


# speedup vs baseline: 2.4844x; 1.0050x over previous
import functools

import jax
import jax.numpy as jnp
from jax import lax
from jax.experimental import pallas as pl
from jax.experimental.pallas import tpu as pltpu
from jax.experimental.pallas import tpu_sc as plsc

N = 10000
E = 320000
D = 128

NC, NS, L = 2, 16, 16
NW = NC * NS
ROWS_PER_W = (2 * E) // NW
CHUNK = 80
N_CHUNKS = ROWS_PER_W // CHUNK


def _sc_gather_mul(srcs, dsts, x):
    mesh = plsc.VectorSubcoreMesh(core_axis_name="c", subcore_axis_name="s")
    f32 = jnp.float32

    @functools.partial(
        pl.kernel,
        out_type=jax.ShapeDtypeStruct((2 * E, D), f32),
        mesh=mesh,
        scratch_types=[
            pltpu.VMEM((ROWS_PER_W,), jnp.int32),
            pltpu.VMEM((ROWS_PER_W,), jnp.int32),
            [pltpu.VMEM((CHUNK, D), f32)] * 2,
            [pltpu.VMEM((CHUNK, D), f32)] * 2,
            [pltpu.VMEM((CHUNK, D), f32)] * 2,
            [pltpu.SemaphoreType.DMA] * 2,
            [pltpu.SemaphoreType.DMA] * 2,
            pltpu.SemaphoreType.DMA,
        ],
    )
    def k(srcs_hbm, dsts_hbm, x_hbm, h_hbm, idx_s, idx_d, a, b, o,
          sem_g, sem_w, sem_i):
        wid = lax.axis_index("s") * NC + lax.axis_index("c")
        w_base = wid * ROWS_PER_W

        ci = pltpu.async_copy(srcs_hbm.at[pl.ds(w_base, ROWS_PER_W)], idx_s,
                              sem_i)
        cd = pltpu.async_copy(dsts_hbm.at[pl.ds(w_base, ROWS_PER_W)], idx_d,
                              sem_i)
        ci.wait()
        cd.wait()

        def fire(kk, p):
            pltpu.async_copy(x_hbm.at[idx_s.at[pl.ds(kk * CHUNK, CHUNK)]],
                             a[p], sem_g[p])
            pltpu.async_copy(x_hbm.at[idx_d.at[pl.ds(kk * CHUNK, CHUNK)]],
                             b[p], sem_g[p])

        fire(0, 0)

        @pl.loop(0, N_CHUNKS, step=2)
        def chunk_pair(k0):
            for p in range(2):
                kk = k0 + p

                @pl.when(kk + 1 < N_CHUNKS)
                def _():
                    fire(kk + 1, 1 - p)

                pltpu.make_async_copy(x_hbm.at[idx_s.at[pl.ds(0, CHUNK)]],
                                      a[p], sem_g[p]).wait()
                pltpu.make_async_copy(x_hbm.at[idx_d.at[pl.ds(0, CHUNK)]],
                                      b[p], sem_g[p]).wait()

                @pl.when(kk >= 2)
                def _():
                    pltpu.make_async_copy(
                        o[p], h_hbm.at[pl.ds(0, CHUNK)],
                        sem_w[p]).wait()

                @plsc.parallel_loop(0, CHUNK, 1, unroll=4)
                def row_body(r):
                    for j in range(D // L):
                        sl = pl.ds(j * L, L)
                        o[p][r, sl] = a[p][r, sl] * b[p][r, sl]

                pltpu.async_copy(
                    o[p], h_hbm.at[pl.ds(w_base + kk * CHUNK, CHUNK)],
                    sem_w[p])

        for p in range(2):
            pltpu.make_async_copy(o[p], h_hbm.at[pl.ds(0, CHUNK)],
                                  sem_w[p]).wait()

    return k(srcs, dsts, x)


B_TC = 12800
NBLK = E // B_TC


def _tc_body(hp_ref, hn_ref, w1_ref, b1_ref, w2_ref, b2_ref, oT_ref):
    cat = jnp.concatenate(
        [jnp.maximum(hp_ref[...], 0.0), jnp.maximum(hn_ref[...], 0.0)], axis=1)
    z = jnp.maximum(
        jnp.dot(cat.astype(jnp.bfloat16), w1_ref[...],
                preferred_element_type=jnp.float32)
        + b1_ref[...], 0.0)
    o = jax.nn.sigmoid(
        jnp.dot(z, w2_ref[...], preferred_element_type=jnp.float32)
        + b2_ref[...])
    oT_ref[...] = o.T


def _tc_mlp(h2, Wb1, bb1, Wb2, bb2):
    return pl.pallas_call(
        _tc_body,
        grid=(NBLK,),
        in_specs=[
            pl.BlockSpec((B_TC, D), lambda i: (i, 0)),
            pl.BlockSpec((B_TC, D), lambda i: (i + NBLK, 0)),
            pl.BlockSpec((2 * D, 3 * D), lambda i: (0, 0)),
            pl.BlockSpec((1, 3 * D), lambda i: (0, 0)),
            pl.BlockSpec((3 * D, 8), lambda i: (0, 0)),
            pl.BlockSpec((1, 8), lambda i: (0, 0)),
        ],
        out_specs=pl.BlockSpec((8, B_TC), lambda i: (0, i)),
        out_shape=jax.ShapeDtypeStruct((8, E), jnp.float32),
    )(h2, h2, Wb1, bb1, Wb2, bb2)


@jax.jit
def kernel(x, edge_index, edge_index_neg, W1, b1, W2, b2, We1, be1, We2, be2):
    srcs = jnp.concatenate([edge_index[0], edge_index_neg[0]])
    dsts = jnp.concatenate([edge_index[1], edge_index_neg[1]])

    h2 = _sc_gather_mul(srcs, dsts, x)

    f32 = jnp.float32
    Wb1 = jnp.zeros((2 * D, 3 * D), f32)
    Wb1 = Wb1.at[:D, :D].set(W1).at[:D, D:2 * D].set(We1).at[D:, 2 * D:].set(We1)
    Wb1 = Wb1.astype(jnp.bfloat16)
    bb1 = jnp.concatenate([b1, be1, be1]).reshape(1, 3 * D)
    Wb2 = jnp.zeros((3 * D, 8), f32)
    Wb2 = Wb2.at[:D, :4].set(W2).at[D:2 * D, 4:5].set(We2).at[2 * D:, 5:6].set(We2)
    bb2 = jnp.concatenate([b2, be2, be2, jnp.zeros((2,), f32)]).reshape(1, 8)

    oT = _tc_mlp(h2, Wb1, bb1, Wb2, bb2)
    return oT[:4].T, oT[4], oT[5]


# speedup vs baseline: 2.6697x; 1.0746x over previous
import functools

import jax
import jax.numpy as jnp
from jax import lax
from jax.experimental import pallas as pl
from jax.experimental.pallas import tpu as pltpu
from jax.experimental.pallas import tpu_sc as plsc

N = 10000
E = 320000
D = 128

NC, NS, L = 2, 16, 16
NW = NC * NS
S = 5
ES = E // S
ROWS_PER_W = (2 * ES) // NW
CHUNK = 80
N_CHUNKS = ROWS_PER_W // CHUNK


def _sc_gather_mul(srcs, dsts, x):
    mesh = plsc.VectorSubcoreMesh(core_axis_name="c", subcore_axis_name="s")
    f32 = jnp.float32

    @functools.partial(
        pl.kernel,
        out_type=jax.ShapeDtypeStruct((2 * ES, D), f32),
        mesh=mesh,
        scratch_types=[
            pltpu.VMEM((ROWS_PER_W,), jnp.int32),
            pltpu.VMEM((ROWS_PER_W,), jnp.int32),
            [pltpu.VMEM((CHUNK, D), f32)] * 2,
            [pltpu.VMEM((CHUNK, D), f32)] * 2,
            [pltpu.VMEM((CHUNK, D), f32)] * 2,
            [pltpu.SemaphoreType.DMA] * 2,
            [pltpu.SemaphoreType.DMA] * 2,
            pltpu.SemaphoreType.DMA,
        ],
    )
    def k(srcs_hbm, dsts_hbm, x_hbm, h_hbm, idx_s, idx_d, a, b, o,
          sem_g, sem_w, sem_i):
        wid = lax.axis_index("s") * NC + lax.axis_index("c")
        w_base = wid * ROWS_PER_W

        ci = pltpu.async_copy(srcs_hbm.at[pl.ds(w_base, ROWS_PER_W)], idx_s,
                              sem_i)
        cd = pltpu.async_copy(dsts_hbm.at[pl.ds(w_base, ROWS_PER_W)], idx_d,
                              sem_i)
        ci.wait()
        cd.wait()

        def fire(kk, p):
            pltpu.async_copy(x_hbm.at[idx_s.at[pl.ds(kk * CHUNK, CHUNK)]],
                             a[p], sem_g[p])
            pltpu.async_copy(x_hbm.at[idx_d.at[pl.ds(kk * CHUNK, CHUNK)]],
                             b[p], sem_g[p])

        fire(0, 0)

        @pl.loop(0, N_CHUNKS, step=2)
        def chunk_pair(k0):
            for p in range(2):
                kk = k0 + p

                @pl.when(kk + 1 < N_CHUNKS)
                def _():
                    fire(kk + 1, 1 - p)

                pltpu.make_async_copy(x_hbm.at[idx_s.at[pl.ds(0, CHUNK)]],
                                      a[p], sem_g[p]).wait()
                pltpu.make_async_copy(x_hbm.at[idx_d.at[pl.ds(0, CHUNK)]],
                                      b[p], sem_g[p]).wait()

                @pl.when(kk >= 2)
                def _():
                    pltpu.make_async_copy(
                        o[p], h_hbm.at[pl.ds(0, CHUNK)],
                        sem_w[p]).wait()

                @plsc.parallel_loop(0, CHUNK, 1, unroll=4)
                def row_body(r):
                    for j in range(D // L):
                        sl = pl.ds(j * L, L)
                        o[p][r, sl] = a[p][r, sl] * b[p][r, sl]

                pltpu.async_copy(
                    o[p], h_hbm.at[pl.ds(w_base + kk * CHUNK, CHUNK)],
                    sem_w[p])

        for p in range(2):
            pltpu.make_async_copy(o[p], h_hbm.at[pl.ds(0, CHUNK)],
                                  sem_w[p]).wait()

    return k(srcs, dsts, x)


B_TC = 12800
NBLK = ES // B_TC


def _tc_body(hp_ref, hn_ref, w1_ref, b1_ref, w2_ref, b2_ref, oT_ref):
    cat = jnp.concatenate(
        [jnp.maximum(hp_ref[...], 0.0), jnp.maximum(hn_ref[...], 0.0)], axis=1)
    z = jnp.maximum(
        jnp.dot(cat.astype(jnp.bfloat16), w1_ref[...],
                preferred_element_type=jnp.float32)
        + b1_ref[...], 0.0)
    o = jax.nn.sigmoid(
        jnp.dot(z, w2_ref[...], preferred_element_type=jnp.float32)
        + b2_ref[...])
    oT_ref[...] = o.T


def _tc_mlp(h2, Wb1, bb1, Wb2, bb2):
    return pl.pallas_call(
        _tc_body,
        grid=(NBLK,),
        in_specs=[
            pl.BlockSpec((B_TC, D), lambda i: (i, 0)),
            pl.BlockSpec((B_TC, D), lambda i: (i + NBLK, 0)),
            pl.BlockSpec((2 * D, 3 * D), lambda i: (0, 0)),
            pl.BlockSpec((1, 3 * D), lambda i: (0, 0)),
            pl.BlockSpec((3 * D, 8), lambda i: (0, 0)),
            pl.BlockSpec((1, 8), lambda i: (0, 0)),
        ],
        out_specs=pl.BlockSpec((8, B_TC), lambda i: (0, i)),
        out_shape=jax.ShapeDtypeStruct((8, ES), jnp.float32),
    )(h2, h2, Wb1, bb1, Wb2, bb2)


@jax.jit
def kernel(x, edge_index, edge_index_neg, W1, b1, W2, b2, We1, be1, We2, be2):
    f32 = jnp.float32
    Wb1 = jnp.zeros((2 * D, 3 * D), f32)
    Wb1 = Wb1.at[:D, :D].set(W1).at[:D, D:2 * D].set(We1).at[D:, 2 * D:].set(We1)
    Wb1 = Wb1.astype(jnp.bfloat16)
    bb1 = jnp.concatenate([b1, be1, be1]).reshape(1, 3 * D)
    Wb2 = jnp.zeros((3 * D, 8), f32)
    Wb2 = Wb2.at[:D, :4].set(W2).at[D:2 * D, 4:5].set(We2).at[2 * D:, 5:6].set(We2)
    bb2 = jnp.concatenate([b2, be2, be2, jnp.zeros((2,), f32)]).reshape(1, 8)

    oTs = []
    for i in range(S):
        sl = slice(i * ES, (i + 1) * ES)
        srcs = jnp.concatenate([edge_index[0, sl], edge_index_neg[0, sl]])
        dsts = jnp.concatenate([edge_index[1, sl], edge_index_neg[1, sl]])
        h2 = _sc_gather_mul(srcs, dsts, x)
        oTs.append(_tc_mlp(h2, Wb1, bb1, Wb2, bb2))
    oT = jnp.concatenate(oTs, axis=1)
    return oT[:4].T, oT[4], oT[5]
